# baseline (speedup 1.0000x reference)
import functools

import jax
import jax.numpy as jnp
from jax import lax
from jax.experimental import pallas as pl
from jax.experimental.pallas import tpu as pltpu
from jax.experimental.pallas import tpu_sc as plsc

N = 10000
E = 320000
F_IN = 128
HID = 64
HEADS = 4
NCLS = 40
NW = 32
EPT = E // NW
ETILE = 16384
RO_PAD = 10024
NB_PAD = 64

_SC_PARAMS = pltpu.CompilerParams(
    use_tc_tiling_on_sc=False, needs_layout_passes=False)
_MESH = dict(core_axis_name="c", subcore_axis_name="s")


def _leaky(z):
    return jnp.where(z > 0, z, 0.2 * z)


def _sc_edge_scores(h, ssrc, sdst, att_flat, C, H):
    CHB = (C // H) // 16
    NCH = EPT // 16

    @functools.partial(
        pl.kernel,
        mesh=plsc.VectorSubcoreMesh(**_MESH),
        compiler_params=_SC_PARAMS,
        out_type=jax.ShapeDtypeStruct((H, E + ETILE + 16), jnp.float32),
        scratch_types=[
            pltpu.VMEM((EPT,), jnp.int32),
            pltpu.VMEM((EPT,), jnp.int32),
            pltpu.VMEM((C,), jnp.float32),
            pltpu.VMEM((2, 16, C), jnp.float32),
            pltpu.VMEM((2, 16, C), jnp.float32),
            pltpu.VMEM((H, EPT), jnp.float32),
            pltpu.SemaphoreType.DMA((2,)),
            pltpu.SemaphoreType.DMA((2,)),
        ],
    )
    def k(h_hbm, src_hbm, dst_hbm, att_hbm, e_hbm,
          src_v, dst_v, att_v, hs_v, hd_v, ebuf, ssem, dsem):
        wid = lax.axis_index("s") * 2 + lax.axis_index("c")
        e0 = wid * EPT
        pltpu.sync_copy(src_hbm.at[pl.ds(e0, EPT)], src_v)
        pltpu.sync_copy(dst_hbm.at[pl.ds(e0, EPT)], dst_v)
        pltpu.sync_copy(att_hbm, att_v)
        lane = lax.iota(jnp.int32, 16)
        ixs = [lane ^ (1 << j) for j in range(4)]
        att_c = [att_v[pl.ds(i * 16, 16)] for i in range(C // 16)]

        def issue(c, par):
            src16 = src_v[pl.ds(c * 16, 16)]
            dst16 = dst_v[pl.ds(c * 16, 16)]
            pltpu.async_copy(h_hbm.at[src16], hs_v.at[par], ssem.at[par])
            pltpu.async_copy(h_hbm.at[dst16], hd_v.at[par], dsem.at[par])

        issue(0, 0)

        def body(c, carry):
            par = lax.rem(c, 2)

            @pl.when(c + 1 < NCH)
            def _():
                issue(c + 1, lax.rem(c + 1, 2))

            pltpu.make_async_copy(
                h_hbm.at[pl.ds(0, 16)], hs_v.at[par], ssem.at[par]).wait()
            pltpu.make_async_copy(
                h_hbm.at[pl.ds(0, 16)], hd_v.at[par], dsem.at[par]).wait()
            evecs = [jnp.zeros((16,), jnp.float32) for _ in range(H)]
            for el in range(16):
                accs = [jnp.zeros((16,), jnp.float32) for _ in range(H)]
                for cb in range(C // 16):
                    hh = cb // CHB
                    hs16 = hs_v[par, el, pl.ds(cb * 16, 16)]
                    hd16 = hd_v[par, el, pl.ds(cb * 16, 16)]
                    accs[hh] = accs[hh] + _leaky(hs16 + hd16) * att_c[cb]
                for hh in range(H):
                    v = accs[hh]
                    for ix in ixs:
                        v = v + jnp.take(v, ix, axis=0)
                    evecs[hh] = jnp.where(lane == el, v, evecs[hh])
            for hh in range(H):
                ebuf[hh, pl.ds(c * 16, 16)] = evecs[hh]
            return carry

        lax.fori_loop(0, NCH, body, jnp.int32(0))
        for hh in range(H):
            pltpu.sync_copy(ebuf.at[hh], e_hbm.at[hh, pl.ds(e0, EPT)])

    return k(h, ssrc, sdst, att_flat)


def _sc_aggregate(h, ssrc_pad, e_pad, ro_pad, nb_pad, C, H):
    CB = C // 16
    CHB = (C // H) // 16

    @functools.partial(
        pl.kernel,
        mesh=plsc.VectorSubcoreMesh(**_MESH),
        compiler_params=_SC_PARAMS,
        out_type=(jax.ShapeDtypeStruct((N, C), jnp.float32),
                  jax.ShapeDtypeStruct((N, C), jnp.float32)),
        scratch_types=[
            pltpu.VMEM((RO_PAD,), jnp.int32),
            pltpu.VMEM((NB_PAD,), jnp.int32),
            pltpu.VMEM((H, ETILE), jnp.float32),
            pltpu.VMEM((ETILE,), jnp.int32),
            pltpu.VMEM((2, 16, C), jnp.float32),
            pltpu.VMEM((C,), jnp.float32),
            pltpu.VMEM((C,), jnp.float32),
            pltpu.SemaphoreType.DMA((2,)),
        ],
    )
    def k(h_hbm, src_hbm, e_hbm, ro_hbm, nb_hbm, out_hbm, noise_hbm,
          ro_v, nb_v, e_v, src_v, rows_v, orow_v, nrow_v, gsem):
        wid = lax.axis_index("s") * 2 + lax.axis_index("c")
        pltpu.sync_copy(ro_hbm, ro_v)
        pltpu.sync_copy(nb_hbm, nb_v)
        n0 = nb_v[pl.ds(wid, 16)][0]
        n1 = nb_v[pl.ds(wid + 1, 16)][0]
        p_lo = ro_v[pl.ds(n0, 16)][0]
        pbase = (p_lo // 16) * 16
        for hh in range(H):
            pltpu.sync_copy(e_hbm.at[hh, pl.ds(pbase, ETILE)], e_v.at[hh])
        pltpu.sync_copy(src_hbm.at[pl.ds(pbase, ETILE)], src_v)
        lane = lax.iota(jnp.int32, 16)
        lanem1 = jnp.maximum(lane - 1, 0)
        seven = jnp.full((16,), 7, jnp.int32)
        NEGV = jnp.full((16,), -jnp.inf, jnp.float32)

        def node_body(n, carry):
            p0 = ro_v[pl.ds(n, 16)][0]
            p1 = ro_v[pl.ds(n + 1, 16)][0]
            q0 = p0 - pbase
            q1 = p1 - pbase
            b0 = q0 // 16
            nblk = lax.div(q1 + 15, 16) - b0

            def issue3(bk, par):
                src16 = src_v[pl.ds((b0 + bk) * 16, 16)]
                pltpu.async_copy(h_hbm.at[src16], rows_v.at[par],
                                 gsem.at[par])

            @pl.when(nblk > 0)
            def _():
                issue3(0, 0)

            def sweep1(bk, cr):
                Ms, dens = cr
                base = (b0 + bk) * 16
                gmask = (base + lane >= q0) & (base + lane < q1)
                newM, newden = [], []
                for hh in range(H):
                    ev = e_v[hh, pl.ds(base, 16)]
                    newden.append(dens[hh] + jnp.sum(
                        jnp.where(gmask, jnp.exp(ev), 0.0), axis=0))
                    evm = jnp.where(gmask, ev, NEGV)
                    s1, _ = plsc.sort_key_val(evm, evm, descending=True)
                    prev = jnp.take(s1, lanem1, axis=0)
                    s1 = jnp.where((s1 == prev) & (lane > 0), NEGV, s1)
                    s2, _ = plsc.sort_key_val(s1, s1, descending=True)
                    un = jnp.maximum(Ms[hh], lax.rev(s2, (0,)))
                    s3, _ = plsc.sort_key_val(un, un, descending=True)
                    prev3 = jnp.take(s3, lanem1, axis=0)
                    s3 = jnp.where((s3 == prev3) & (lane > 0), NEGV, s3)
                    s4, _ = plsc.sort_key_val(s3, s3, descending=True)
                    newM.append(s4)
                return (tuple(newM), tuple(newden))

            Ms, denf = lax.fori_loop(
                0, nblk, sweep1,
                (tuple(NEGV for _ in range(H)),
                 tuple(jnp.float32(0.0) for _ in range(H))))
            ts = [jnp.take(Ms[hh], seven, axis=0) for hh in range(H)]

            def sweep2(bk, dm):
                base = (b0 + bk) * 16
                gmask = (base + lane >= q0) & (base + lane < q1)
                out = []
                for hh in range(H):
                    ev = e_v[hh, pl.ds(base, 16)]
                    out.append(dm[hh] + jnp.sum(
                        jnp.where(gmask & (ev >= ts[hh]), jnp.exp(ev), 0.0),
                        axis=0))
                return tuple(out)

            denm = lax.fori_loop(
                0, nblk, sweep2, tuple(jnp.float32(0.0) for _ in range(H)))
            one16 = jnp.ones((16,), jnp.float32)
            invf = [one16 / (jnp.full((16,), denf[hh], jnp.float32) + 1e-16)
                    for hh in range(H)]
            invm = [one16 / (jnp.full((16,), denm[hh], jnp.float32) + 1e-16)
                    for hh in range(H)]

            def sweep3(bk, accs):
                acc_o, acc_n = accs
                base = (b0 + bk) * 16
                par = lax.rem(bk, 2)

                @pl.when(bk + 1 < nblk)
                def _():
                    issue3(bk + 1, lax.rem(bk + 1, 2))

                pltpu.make_async_copy(
                    h_hbm.at[pl.ds(0, 16)], rows_v.at[par],
                    gsem.at[par]).wait()
                gmask = (base + lane >= q0) & (base + lane < q1)
                wf, wm = [], []
                for hh in range(H):
                    ev = e_v[hh, pl.ds(base, 16)]
                    ex = jnp.exp(ev)
                    wf.append(jnp.where(gmask, ex, 0.0) * invf[hh])
                    wm.append(jnp.where(gmask & (ev >= ts[hh]), ex, 0.0)
                              * invm[hh])
                acc_o, acc_n = list(acc_o), list(acc_n)
                for el in range(16):
                    el16 = jnp.full((16,), el, jnp.int32)
                    wfe = [jnp.take(wf[hh], el16, axis=0) for hh in range(H)]
                    wme = [jnp.take(wm[hh], el16, axis=0) for hh in range(H)]
                    for cb in range(CB):
                        hh = cb // CHB
                        rv = rows_v[par, el, pl.ds(cb * 16, 16)]
                        acc_o[cb] = acc_o[cb] + wme[hh] * rv
                        acc_n[cb] = acc_n[cb] + wfe[hh] * rv
                return (tuple(acc_o), tuple(acc_n))

            zero16 = jnp.zeros((16,), jnp.float32)
            acc_o, acc_n = lax.fori_loop(
                0, nblk, sweep3,
                (tuple(zero16 for _ in range(CB)),
                 tuple(zero16 for _ in range(CB))))
            for cb in range(CB):
                orow_v[pl.ds(cb * 16, 16)] = acc_o[cb]
                nrow_v[pl.ds(cb * 16, 16)] = acc_n[cb]
            pltpu.sync_copy(orow_v, out_hbm.at[n])
            pltpu.sync_copy(nrow_v, noise_hbm.at[n])
            return carry

        lax.fori_loop(n0, n1, node_body, jnp.int32(0))

    return k(h, ssrc_pad, e_pad, ro_pad, nb_pad)


def _tc_mm1(x, W1):
    def body(x_ref, w_ref, o_ref):
        o_ref[...] = jnp.dot(x_ref[...], w_ref[...],
                             preferred_element_type=jnp.float32)

    return pl.pallas_call(
        body,
        grid=(10,),
        in_specs=[pl.BlockSpec((1000, F_IN), lambda i: (i, 0)),
                  pl.BlockSpec((F_IN, HEADS * HID), lambda i: (0, 0))],
        out_specs=pl.BlockSpec((1000, HEADS * HID), lambda i: (i, 0)),
        out_shape=jax.ShapeDtypeStruct((N, HEADS * HID), jnp.float32),
    )(x, W1)


def _elu(v):
    return jnp.where(v > 0, v, jnp.exp(v) - 1.0)


def _logsm(v):
    m = jnp.max(v, axis=-1, keepdims=True)
    return v - m - jnp.log(jnp.sum(jnp.exp(v - m), axis=-1, keepdims=True))


def _tc_mid(out1, noise1, b1, W2p, l1W, l1b, l2W, l2b):
    def body(o1, nz1, b1r, w2r, l1wr, l1br, l2wr, l2br, h2o, n1o):
        x1 = _elu(o1[...] + b1r[...])
        h2o[...] = jnp.dot(x1, w2r[...], preferred_element_type=jnp.float32)
        nz = nz1[...] + b1r[...]
        t = _elu(jnp.dot(nz, l1wr[...], preferred_element_type=jnp.float32)
                 + l1br[...])
        n1 = jnp.dot(t, l2wr[...], preferred_element_type=jnp.float32) \
            + l2br[...]
        n1o[...] = _logsm(n1)

    D = HEADS * HID
    return pl.pallas_call(
        body,
        grid=(10,),
        in_specs=[pl.BlockSpec((1000, D), lambda i: (i, 0)),
                  pl.BlockSpec((1000, D), lambda i: (i, 0)),
                  pl.BlockSpec((1, D), lambda i: (0, 0)),
                  pl.BlockSpec((D, 64), lambda i: (0, 0)),
                  pl.BlockSpec((D, 128), lambda i: (0, 0)),
                  pl.BlockSpec((1, 128), lambda i: (0, 0)),
                  pl.BlockSpec((128, NCLS), lambda i: (0, 0)),
                  pl.BlockSpec((1, NCLS), lambda i: (0, 0))],
        out_specs=(pl.BlockSpec((1000, 64), lambda i: (i, 0)),
                   pl.BlockSpec((1000, NCLS), lambda i: (i, 0))),
        out_shape=(jax.ShapeDtypeStruct((N, 64), jnp.float32),
                   jax.ShapeDtypeStruct((N, NCLS), jnp.float32)),
    )(out1, noise1, b1, W2p, l1W, l1b, l2W, l2b)


def _tc_fin(out2, noise2, b2):
    def body(o2, nz2, b2r, xo, no):
        xo[...] = _logsm(o2[...][:, :NCLS] + b2r[...])
        no[...] = _logsm(nz2[...][:, :NCLS] + b2r[...])

    return pl.pallas_call(
        body,
        grid=(10,),
        in_specs=[pl.BlockSpec((1000, 64), lambda i: (i, 0)),
                  pl.BlockSpec((1000, 64), lambda i: (i, 0)),
                  pl.BlockSpec((1, NCLS), lambda i: (0, 0))],
        out_specs=(pl.BlockSpec((1000, NCLS), lambda i: (i, 0)),
                   pl.BlockSpec((1000, NCLS), lambda i: (i, 0))),
        out_shape=(jax.ShapeDtypeStruct((N, NCLS), jnp.float32),
                   jax.ShapeDtypeStruct((N, NCLS), jnp.float32)),
    )(out2, noise2, b2)


def kernel(x, edge_index, W1, att1, b1, W2, att2, b2,
           lin1_W, lin1_b, lin2_W, lin2_b):
    src = edge_index[0]
    dst = edge_index[1]
    sdst, ssrc = lax.sort((dst.astype(jnp.int32), src.astype(jnp.int32)),
                          num_keys=1)
    ro = jnp.searchsorted(sdst, jnp.arange(N + 1), side="left") \
        .astype(jnp.int32)
    nbounds = jnp.searchsorted(ro, jnp.arange(NW + 1) * EPT, side="left") \
        .astype(jnp.int32)
    nbounds = nbounds.at[NW].set(N)
    ro_pad = jnp.pad(ro, (0, RO_PAD - (N + 1)), constant_values=E)
    nb_pad = jnp.pad(nbounds, (0, NB_PAD - (NW + 1)))
    ssrc_pad = jnp.pad(ssrc, (0, ETILE))

    h1 = _tc_mm1(x, W1)
    e1 = _sc_edge_scores(h1, ssrc, sdst, att1.reshape(-1), HEADS * HID, HEADS)
    out1, noise1 = _sc_aggregate(h1, ssrc_pad, e1, ro_pad, nb_pad,
                                 HEADS * HID, HEADS)

    W2p = jnp.pad(W2, ((0, 0), (0, 64 - NCLS)))
    h2, n1_lsm = _tc_mid(out1, noise1, b1.reshape(1, -1), W2p,
                         lin1_W, lin1_b.reshape(1, -1),
                         lin2_W, lin2_b.reshape(1, -1))

    att2p = jnp.pad(att2.reshape(-1), (0, 64 - NCLS))
    e2 = _sc_edge_scores(h2, ssrc, sdst, att2p, 64, 1)
    out2, noise2 = _sc_aggregate(h2, ssrc_pad, e2, ro_pad, nb_pad, 64, 1)

    x2_lsm, n2_lsm = _tc_fin(out2, noise2, b2.reshape(1, -1))
    return (x2_lsm, n1_lsm, n2_lsm)


# speedup vs baseline: 1.0010x; 1.0010x over previous
import functools

import jax
import jax.numpy as jnp
from jax import lax
from jax.experimental import pallas as pl
from jax.experimental.pallas import tpu as pltpu
from jax.experimental.pallas import tpu_sc as plsc

N = 10000
E = 320000
F_IN = 128
HID = 64
HEADS = 4
NCLS = 40
NW = 32
EPT = E // NW
ETILE = 16384
RO_PAD = 10024
NB_PAD = 64

_SC_PARAMS = pltpu.CompilerParams(
    use_tc_tiling_on_sc=False, needs_layout_passes=False)
_MESH = dict(core_axis_name="c", subcore_axis_name="s")


def _leaky(z):
    return jnp.where(z > 0, z, 0.2 * z)


def _sc_edge_scores(h, ssrc, sdst, att_flat, C, H):
    CHB = (C // H) // 16
    NCH = EPT // 16

    @functools.partial(
        pl.kernel,
        mesh=plsc.VectorSubcoreMesh(**_MESH),
        compiler_params=_SC_PARAMS,
        out_type=jax.ShapeDtypeStruct((H, E + ETILE + 16), jnp.float32),
        scratch_types=[
            pltpu.VMEM((EPT,), jnp.int32),
            pltpu.VMEM((EPT,), jnp.int32),
            pltpu.VMEM((C,), jnp.float32),
            pltpu.VMEM((2, 16, C), jnp.float32),
            pltpu.VMEM((2, 16, C), jnp.float32),
            pltpu.VMEM((H, EPT), jnp.float32),
            pltpu.SemaphoreType.DMA((2,)),
            pltpu.SemaphoreType.DMA((2,)),
        ],
    )
    def k(h_hbm, src_hbm, dst_hbm, att_hbm, e_hbm,
          src_v, dst_v, att_v, hs_v, hd_v, ebuf, ssem, dsem):
        wid = lax.axis_index("s") * 2 + lax.axis_index("c")
        e0 = wid * EPT
        pltpu.sync_copy(src_hbm.at[pl.ds(e0, EPT)], src_v)
        pltpu.sync_copy(dst_hbm.at[pl.ds(e0, EPT)], dst_v)
        pltpu.sync_copy(att_hbm, att_v)
        lane = lax.iota(jnp.int32, 16)
        att_c = [att_v[pl.ds(i * 16, 16)] for i in range(C // 16)]

        def issue(c, par):
            src16 = src_v[pl.ds(c * 16, 16)]
            dst16 = dst_v[pl.ds(c * 16, 16)]
            pltpu.async_copy(h_hbm.at[src16], hs_v.at[par], ssem.at[par])
            pltpu.async_copy(h_hbm.at[dst16], hd_v.at[par], dsem.at[par])

        issue(0, 0)

        def body(c, carry):
            par = lax.rem(c, 2)

            @pl.when(c + 1 < NCH)
            def _():
                issue(c + 1, lax.rem(c + 1, 2))

            pltpu.make_async_copy(
                h_hbm.at[pl.ds(0, 16)], hs_v.at[par], ssem.at[par]).wait()
            pltpu.make_async_copy(
                h_hbm.at[pl.ds(0, 16)], hd_v.at[par], dsem.at[par]).wait()
            evecs = [jnp.zeros((16,), jnp.float32) for _ in range(H)]
            for el in range(16):
                accs = [jnp.zeros((16,), jnp.float32) for _ in range(H)]
                for cb in range(C // 16):
                    hh = cb // CHB
                    hs16 = hs_v[par, el, pl.ds(cb * 16, 16)]
                    hd16 = hd_v[par, el, pl.ds(cb * 16, 16)]
                    accs[hh] = accs[hh] + _leaky(hs16 + hd16) * att_c[cb]
                for hh in range(H):
                    s = jnp.sum(accs[hh], axis=0)
                    evecs[hh] = jnp.where(lane == el, s, evecs[hh])
            for hh in range(H):
                ebuf[hh, pl.ds(c * 16, 16)] = evecs[hh]
            return carry

        lax.fori_loop(0, NCH, body, jnp.int32(0))
        for hh in range(H):
            pltpu.sync_copy(ebuf.at[hh], e_hbm.at[hh, pl.ds(e0, EPT)])

    return k(h, ssrc, sdst, att_flat)


def _sc_aggregate(h, ssrc_pad, e_pad, ro_pad, nb_pad, C, H):
    CB = C // 16
    CHB = (C // H) // 16

    @functools.partial(
        pl.kernel,
        mesh=plsc.VectorSubcoreMesh(**_MESH),
        compiler_params=_SC_PARAMS,
        out_type=(jax.ShapeDtypeStruct((N, C), jnp.float32),
                  jax.ShapeDtypeStruct((N, C), jnp.float32)),
        scratch_types=[
            pltpu.VMEM((RO_PAD,), jnp.int32),
            pltpu.VMEM((NB_PAD,), jnp.int32),
            pltpu.VMEM((H, ETILE), jnp.float32),
            pltpu.VMEM((ETILE,), jnp.int32),
            pltpu.VMEM((2, 16, C), jnp.float32),
            pltpu.VMEM((C,), jnp.float32),
            pltpu.VMEM((C,), jnp.float32),
            pltpu.SemaphoreType.DMA((2,)),
        ],
    )
    def k(h_hbm, src_hbm, e_hbm, ro_hbm, nb_hbm, out_hbm, noise_hbm,
          ro_v, nb_v, e_v, src_v, rows_v, orow_v, nrow_v, gsem):
        wid = lax.axis_index("s") * 2 + lax.axis_index("c")
        pltpu.sync_copy(ro_hbm, ro_v)
        pltpu.sync_copy(nb_hbm, nb_v)
        n0 = nb_v[pl.ds(wid, 16)][0]
        n1 = nb_v[pl.ds(wid + 1, 16)][0]
        p_lo = ro_v[pl.ds(n0, 16)][0]
        pbase = (p_lo // 16) * 16
        for hh in range(H):
            pltpu.sync_copy(e_hbm.at[hh, pl.ds(pbase, ETILE)], e_v.at[hh])
        pltpu.sync_copy(src_hbm.at[pl.ds(pbase, ETILE)], src_v)
        lane = lax.iota(jnp.int32, 16)
        lanem1 = jnp.maximum(lane - 1, 0)
        seven = jnp.full((16,), 7, jnp.int32)
        NEGV = jnp.full((16,), -jnp.inf, jnp.float32)

        def node_body(n, carry):
            p0 = ro_v[pl.ds(n, 16)][0]
            p1 = ro_v[pl.ds(n + 1, 16)][0]
            q0 = p0 - pbase
            q1 = p1 - pbase
            b0 = q0 // 16
            nblk = lax.div(q1 + 15, 16) - b0

            def issue3(bk, par):
                src16 = src_v[pl.ds((b0 + bk) * 16, 16)]
                pltpu.async_copy(h_hbm.at[src16], rows_v.at[par],
                                 gsem.at[par])

            @pl.when(nblk > 0)
            def _():
                issue3(0, 0)

            def sweep1(bk, cr):
                Ms, dens = cr
                base = (b0 + bk) * 16
                gmask = (base + lane >= q0) & (base + lane < q1)
                newM, newden = [], []
                for hh in range(H):
                    ev = e_v[hh, pl.ds(base, 16)]
                    newden.append(dens[hh] + jnp.sum(
                        jnp.where(gmask, jnp.exp(ev), 0.0), axis=0))
                    evm = jnp.where(gmask, ev, NEGV)
                    s1, _ = plsc.sort_key_val(evm, evm, descending=True)
                    prev = jnp.take(s1, lanem1, axis=0)
                    s1 = jnp.where((s1 == prev) & (lane > 0), NEGV, s1)
                    s2, _ = plsc.sort_key_val(s1, s1, descending=True)
                    un = jnp.maximum(Ms[hh], lax.rev(s2, (0,)))
                    s3, _ = plsc.sort_key_val(un, un, descending=True)
                    prev3 = jnp.take(s3, lanem1, axis=0)
                    s3 = jnp.where((s3 == prev3) & (lane > 0), NEGV, s3)
                    s4, _ = plsc.sort_key_val(s3, s3, descending=True)
                    newM.append(s4)
                return (tuple(newM), tuple(newden))

            Ms, denf = lax.fori_loop(
                0, nblk, sweep1,
                (tuple(NEGV for _ in range(H)),
                 tuple(jnp.float32(0.0) for _ in range(H))))
            ts = [jnp.take(Ms[hh], seven, axis=0) for hh in range(H)]

            def sweep2(bk, dm):
                base = (b0 + bk) * 16
                gmask = (base + lane >= q0) & (base + lane < q1)
                out = []
                for hh in range(H):
                    ev = e_v[hh, pl.ds(base, 16)]
                    out.append(dm[hh] + jnp.sum(
                        jnp.where(gmask & (ev >= ts[hh]), jnp.exp(ev), 0.0),
                        axis=0))
                return tuple(out)

            denm = lax.fori_loop(
                0, nblk, sweep2, tuple(jnp.float32(0.0) for _ in range(H)))
            one16 = jnp.ones((16,), jnp.float32)
            invf = [one16 / (jnp.full((16,), denf[hh], jnp.float32) + 1e-16)
                    for hh in range(H)]
            invm = [one16 / (jnp.full((16,), denm[hh], jnp.float32) + 1e-16)
                    for hh in range(H)]

            def sweep3(bk, accs):
                acc_o, acc_n = accs
                base = (b0 + bk) * 16
                par = lax.rem(bk, 2)

                @pl.when(bk + 1 < nblk)
                def _():
                    issue3(bk + 1, lax.rem(bk + 1, 2))

                pltpu.make_async_copy(
                    h_hbm.at[pl.ds(0, 16)], rows_v.at[par],
                    gsem.at[par]).wait()
                gmask = (base + lane >= q0) & (base + lane < q1)
                wf, wm = [], []
                for hh in range(H):
                    ev = e_v[hh, pl.ds(base, 16)]
                    ex = jnp.exp(ev)
                    wf.append(jnp.where(gmask, ex, 0.0) * invf[hh])
                    wm.append(jnp.where(gmask & (ev >= ts[hh]), ex, 0.0)
                              * invm[hh])
                acc_o, acc_n = list(acc_o), list(acc_n)
                for el in range(16):
                    el16 = jnp.full((16,), el, jnp.int32)
                    wfe = [jnp.take(wf[hh], el16, axis=0) for hh in range(H)]
                    wme = [jnp.take(wm[hh], el16, axis=0) for hh in range(H)]
                    for cb in range(CB):
                        hh = cb // CHB
                        rv = rows_v[par, el, pl.ds(cb * 16, 16)]
                        acc_o[cb] = acc_o[cb] + wme[hh] * rv
                        acc_n[cb] = acc_n[cb] + wfe[hh] * rv
                return (tuple(acc_o), tuple(acc_n))

            zero16 = jnp.zeros((16,), jnp.float32)
            acc_o, acc_n = lax.fori_loop(
                0, nblk, sweep3,
                (tuple(zero16 for _ in range(CB)),
                 tuple(zero16 for _ in range(CB))))
            for cb in range(CB):
                orow_v[pl.ds(cb * 16, 16)] = acc_o[cb]
                nrow_v[pl.ds(cb * 16, 16)] = acc_n[cb]
            pltpu.sync_copy(orow_v, out_hbm.at[n])
            pltpu.sync_copy(nrow_v, noise_hbm.at[n])
            return carry

        lax.fori_loop(n0, n1, node_body, jnp.int32(0))

    return k(h, ssrc_pad, e_pad, ro_pad, nb_pad)


def _tc_mm1(x, W1):
    def body(x_ref, w_ref, o_ref):
        o_ref[...] = jnp.dot(x_ref[...], w_ref[...],
                             preferred_element_type=jnp.float32)

    return pl.pallas_call(
        body,
        grid=(10,),
        in_specs=[pl.BlockSpec((1000, F_IN), lambda i: (i, 0)),
                  pl.BlockSpec((F_IN, HEADS * HID), lambda i: (0, 0))],
        out_specs=pl.BlockSpec((1000, HEADS * HID), lambda i: (i, 0)),
        out_shape=jax.ShapeDtypeStruct((N, HEADS * HID), jnp.float32),
    )(x, W1)


def _elu(v):
    return jnp.where(v > 0, v, jnp.exp(v) - 1.0)


def _logsm(v):
    m = jnp.max(v, axis=-1, keepdims=True)
    return v - m - jnp.log(jnp.sum(jnp.exp(v - m), axis=-1, keepdims=True))


def _tc_mid(out1, noise1, b1, W2p, l1W, l1b, l2W, l2b):
    def body(o1, nz1, b1r, w2r, l1wr, l1br, l2wr, l2br, h2o, n1o):
        x1 = _elu(o1[...] + b1r[...])
        h2o[...] = jnp.dot(x1, w2r[...], preferred_element_type=jnp.float32)
        nz = nz1[...] + b1r[...]
        t = _elu(jnp.dot(nz, l1wr[...], preferred_element_type=jnp.float32)
                 + l1br[...])
        n1 = jnp.dot(t, l2wr[...], preferred_element_type=jnp.float32) \
            + l2br[...]
        n1o[...] = _logsm(n1)

    D = HEADS * HID
    return pl.pallas_call(
        body,
        grid=(10,),
        in_specs=[pl.BlockSpec((1000, D), lambda i: (i, 0)),
                  pl.BlockSpec((1000, D), lambda i: (i, 0)),
                  pl.BlockSpec((1, D), lambda i: (0, 0)),
                  pl.BlockSpec((D, 64), lambda i: (0, 0)),
                  pl.BlockSpec((D, 128), lambda i: (0, 0)),
                  pl.BlockSpec((1, 128), lambda i: (0, 0)),
                  pl.BlockSpec((128, NCLS), lambda i: (0, 0)),
                  pl.BlockSpec((1, NCLS), lambda i: (0, 0))],
        out_specs=(pl.BlockSpec((1000, 64), lambda i: (i, 0)),
                   pl.BlockSpec((1000, NCLS), lambda i: (i, 0))),
        out_shape=(jax.ShapeDtypeStruct((N, 64), jnp.float32),
                   jax.ShapeDtypeStruct((N, NCLS), jnp.float32)),
    )(out1, noise1, b1, W2p, l1W, l1b, l2W, l2b)


def _tc_fin(out2, noise2, b2):
    def body(o2, nz2, b2r, xo, no):
        xo[...] = _logsm(o2[...][:, :NCLS] + b2r[...])
        no[...] = _logsm(nz2[...][:, :NCLS] + b2r[...])

    return pl.pallas_call(
        body,
        grid=(10,),
        in_specs=[pl.BlockSpec((1000, 64), lambda i: (i, 0)),
                  pl.BlockSpec((1000, 64), lambda i: (i, 0)),
                  pl.BlockSpec((1, NCLS), lambda i: (0, 0))],
        out_specs=(pl.BlockSpec((1000, NCLS), lambda i: (i, 0)),
                   pl.BlockSpec((1000, NCLS), lambda i: (i, 0))),
        out_shape=(jax.ShapeDtypeStruct((N, NCLS), jnp.float32),
                   jax.ShapeDtypeStruct((N, NCLS), jnp.float32)),
    )(out2, noise2, b2)


def kernel(x, edge_index, W1, att1, b1, W2, att2, b2,
           lin1_W, lin1_b, lin2_W, lin2_b):
    src = edge_index[0]
    dst = edge_index[1]
    sdst, ssrc = lax.sort((dst.astype(jnp.int32), src.astype(jnp.int32)),
                          num_keys=1)
    ro = jnp.searchsorted(sdst, jnp.arange(N + 1), side="left") \
        .astype(jnp.int32)
    nbounds = jnp.searchsorted(ro, jnp.arange(NW + 1) * EPT, side="left") \
        .astype(jnp.int32)
    nbounds = nbounds.at[NW].set(N)
    ro_pad = jnp.pad(ro, (0, RO_PAD - (N + 1)), constant_values=E)
    nb_pad = jnp.pad(nbounds, (0, NB_PAD - (NW + 1)))
    ssrc_pad = jnp.pad(ssrc, (0, ETILE))

    h1 = _tc_mm1(x, W1)
    e1 = _sc_edge_scores(h1, ssrc, sdst, att1.reshape(-1), HEADS * HID, HEADS)
    out1, noise1 = _sc_aggregate(h1, ssrc_pad, e1, ro_pad, nb_pad,
                                 HEADS * HID, HEADS)

    W2p = jnp.pad(W2, ((0, 0), (0, 64 - NCLS)))
    h2, n1_lsm = _tc_mid(out1, noise1, b1.reshape(1, -1), W2p,
                         lin1_W, lin1_b.reshape(1, -1),
                         lin2_W, lin2_b.reshape(1, -1))

    att2p = jnp.pad(att2.reshape(-1), (0, 64 - NCLS))
    e2 = _sc_edge_scores(h2, ssrc, sdst, att2p, 64, 1)
    out2, noise2 = _sc_aggregate(h2, ssrc_pad, e2, ro_pad, nb_pad, 64, 1)

    x2_lsm, n2_lsm = _tc_fin(out2, noise2, b2.reshape(1, -1))
    return (x2_lsm, n1_lsm, n2_lsm)
